# baseline (speedup 1.0000x reference)
import functools

import jax
import jax.numpy as jnp
from jax import lax
from jax.experimental import pallas as pl
from jax.experimental.pallas import tpu as pltpu
from jax.experimental.pallas import tpu_sc as plsc

N = 100000
C = 32
SUMNP = 1600000
NCLUST = 4096
FULLSCALE = 14.0
SCALE = 50.0

NW = 32
CPW = NCLUST // NW
SPW = SUMNP // NW
CH = 128
W2 = 200
NST = SPW // W2
BIG = 3.0e38

_mesh = plsc.VectorSubcoreMesh(core_axis_name="c", subcore_axis_name="s")
_cparams = pltpu.CompilerParams(use_tc_tiling_on_sc=False,
                                needs_layout_passes=False)


def _wid():
    return lax.axis_index("s") * 2 + lax.axis_index("c")


def _stats_body(pidx_hbm, coords_hbm, off_hbm, stats_hbm,
                off_v, idx_v, rows_v, stats_v, sem):
    w = _wid()
    cbase = w * CPW
    pltpu.sync_copy(off_hbm.at[pl.ds(cbase, CPW + 16)], off_v)

    @pl.loop(0, CPW)
    def _cluster(c):
        vv = off_v[pl.ds(c, 16)]
        start = vv[0]
        end = vv[1]
        s_al = (start // 8) * 8
        nch = (end - s_al + CH - 1) // CH

        def chunk(k, carry):
            acc_s, acc_mn, acc_mx = carry
            pos = s_al + k * CH
            pltpu.sync_copy(pidx_hbm.at[pl.ds(pos, CH)], idx_v)
            pltpu.async_copy(coords_hbm.at[idx_v], rows_v, sem).wait()
            lo = jnp.maximum(start - pos, 0)
            hi = jnp.minimum(end - pos, CH)

            def point(j, pc):
                s2, mn2, mx2 = pc
                v = rows_v[j]
                return (s2 + v, jnp.minimum(mn2, v), jnp.maximum(mx2, v))

            return lax.fori_loop(lo, hi, point, (acc_s, acc_mn, acc_mx))

        init = (jnp.zeros((16,), jnp.float32),
                jnp.full((16,), BIG, jnp.float32),
                jnp.full((16,), -BIG, jnp.float32))
        acc_s, acc_mn, acc_mx = lax.fori_loop(0, nch, chunk, init)
        stats_v[c, pl.ds(0, 16)] = acc_s
        stats_v[c, pl.ds(16, 16)] = acc_mn
        stats_v[c, pl.ds(32, 16)] = acc_mx

    pltpu.sync_copy(stats_v, stats_hbm.at[pl.ds(cbase, CPW)])


@jax.jit
def _sc_stats(pidx_pad, coords_pad, off_pad):
    k = pl.kernel(
        _stats_body,
        out_type=jax.ShapeDtypeStruct((NCLUST, 48), jnp.float32),
        mesh=_mesh,
        scratch_types=[
            pltpu.VMEM((CPW + 16,), jnp.int32),
            pltpu.VMEM((CH,), jnp.int32),
            pltpu.VMEM((CH, 16), jnp.float32),
            pltpu.VMEM((CPW, 48), jnp.float32),
            pltpu.SemaphoreType.DMA,
        ],
        compiler_params=_cparams,
    )
    return k(pidx_pad, coords_pad, off_pad)


def _params_body(stats_ref, lo_ref, hi_ref, r_ref, out_ref):
    stats = stats_ref[...]
    cnt = jnp.maximum((hi_ref[...] - lo_ref[...]).astype(jnp.float32), 1.0)
    mean = stats[:, 0:3] / cnt
    cmin = stats[:, 16:19] - mean
    cmax = stats[:, 32:35] - mean
    ext = jnp.max(cmax - cmin, axis=1, keepdims=True)
    scale = jnp.minimum(1.0 / jnp.maximum(ext / FULLSCALE, 1e-6) - 0.01, SCALE)
    min_xyz = cmin * scale
    max_xyz = cmax * scale
    rng = max_xyz - min_xyz
    r0 = r_ref[0:1, :]
    r1 = r_ref[1:2, :]
    off = (-min_xyz
           + jnp.maximum(FULLSCALE - rng - 0.001, 0.0) * r0
           + jnp.minimum(FULLSCALE - rng + 0.001, 0.0) * r1)
    b = off - mean * scale
    mul = jnp.concatenate(
        [jnp.broadcast_to(scale, (NCLUST, 3)),
         jnp.ones((NCLUST, 13), jnp.float32)], axis=1)
    add = jnp.concatenate([b, jnp.zeros((NCLUST, 13), jnp.float32)], axis=1)
    out_ref[...] = jnp.concatenate([mul, add], axis=1)


@jax.jit
def _tc_params(stats, off_lo, off_hi, r):
    return pl.pallas_call(
        _params_body,
        out_shape=jax.ShapeDtypeStruct((NCLUST, 32), jnp.float32),
    )(stats, off_lo, off_hi, r)


_QCHUNKS = ((0, 128), (128, 72))


def _apply_body(pidx_hbm, cid_hbm, t_hbm, p_hbm, out_hbm,
                idx0, idx1, cid0, cid1, g0, g1, p0, p1, o0, o1,
                sg0, sg1, sp0, sp1, so0, so1):
    idx = (idx0, idx1)
    cidv = (cid0, cid1)
    g = (g0, g1)
    p = (p0, p1)
    o = (o0, o1)
    sg = (sg0, sg1)
    sp = (sp0, sp1)
    so = (so0, so1)
    w = _wid()
    base0 = w * SPW
    lane = lax.iota(jnp.int32, 16)

    def fetch(s, b):
        base = base0 + s * W2
        pltpu.sync_copy(pidx_hbm.at[pl.ds(base, W2)], idx[b])
        pltpu.sync_copy(cid_hbm.at[pl.ds(base, W2)], cidv[b])
        for q0, qn in _QCHUNKS:
            pltpu.async_copy(t_hbm.at[idx[b].at[pl.ds(q0, qn)]],
                             g[b].at[pl.ds(q0, qn)], sg[b])
            pltpu.async_copy(p_hbm.at[cidv[b].at[pl.ds(q0, qn)]],
                             p[b].at[pl.ds(q0, qn)], sp[b])

    def drain(b):
        for q0, qn in _QCHUNKS:
            pltpu.make_async_copy(t_hbm.at[idx[b].at[pl.ds(q0, qn)]],
                                  g[b].at[pl.ds(q0, qn)], sg[b]).wait()
            pltpu.make_async_copy(p_hbm.at[cidv[b].at[pl.ds(q0, qn)]],
                                  p[b].at[pl.ds(q0, qn)], sp[b]).wait()

    def compute(b):
        gv = g[b]
        pv = p[b]
        ov = o[b]

        @plsc.parallel_loop(0, W2, unroll=4)
        def _pt(j):
            r0 = gv[j, pl.ds(0, 16)]
            m = pv[j, pl.ds(0, 16)]
            a = pv[j, pl.ds(16, 16)]
            ov[pl.ds(j * 35, 16)] = r0 * m + a
            ov[pl.ds(j * 35 + 16, 16)] = gv[j, pl.ds(16, 16)]
            plsc.store_scatter(ov, [j * 35 + 32 + lane],
                               gv[j, pl.ds(32, 16)], mask=lane < 3)

    def out_issue(s, b):
        base = base0 + s * W2
        return pltpu.async_copy(o[b].at[pl.ds(0, W2 * 35)],
                                out_hbm.at[pl.ds(base * 35, W2 * 35)], so[b])

    fetch(0, 0)
    fetch(1, 1)

    @pl.loop(0, NST // 2)
    def _pair(pr):
        s0 = 2 * pr
        drain(0)
        compute(0)
        cp0 = out_issue(s0, 0)
        fetch(s0 + 2, 0)
        drain(1)
        compute(1)
        cp1 = out_issue(s0 + 1, 1)
        fetch(s0 + 3, 1)
        cp0.wait()
        cp1.wait()

    drain(0)
    drain(1)


@jax.jit
def _sc_apply(pidx, cid, t, params):
    k = pl.kernel(
        _apply_body,
        out_type=jax.ShapeDtypeStruct((SUMNP * 35,), jnp.float32),
        mesh=_mesh,
        scratch_types=(
            [pltpu.VMEM((W2,), jnp.int32)] * 4
            + [pltpu.VMEM((W2, 48), jnp.float32),
               pltpu.VMEM((W2, 48), jnp.float32),
               pltpu.VMEM((W2, 32), jnp.float32),
               pltpu.VMEM((W2, 32), jnp.float32),
               pltpu.VMEM((W2 * 35 + 16,), jnp.float32),
               pltpu.VMEM((W2 * 35 + 16,), jnp.float32)]
            + [pltpu.SemaphoreType.DMA] * 6
        ),
        compiler_params=_cparams,
    )
    return k(pidx, cid, t, params)


def kernel(feats, coords, cluster_ids, point_idxs, clusters_offset):
    z13 = jnp.zeros((N, 13), jnp.float32)
    coords_pad = jnp.concatenate([coords, z13], axis=1)
    t = jnp.concatenate([coords, feats, z13], axis=1)
    pidx = point_idxs.astype(jnp.int32)
    cid = cluster_ids.astype(jnp.int32)
    offs = clusters_offset.astype(jnp.int32)
    pidx_pad = jnp.concatenate([pidx, jnp.zeros((512,), jnp.int32)])
    cid_pad = jnp.concatenate([cid, jnp.zeros((512,), jnp.int32)])
    off_pad = jnp.concatenate(
        [offs, jnp.full((15,), SUMNP, jnp.int32)])

    stats = _sc_stats(pidx_pad, coords_pad, off_pad)
    r = jax.random.uniform(jax.random.key(42), (2, 3), dtype=jnp.float32)
    params = _tc_params(stats,
                        offs[:-1].reshape(NCLUST, 1),
                        offs[1:].reshape(NCLUST, 1), r)
    return _sc_apply(pidx_pad, cid_pad, t, params).reshape(SUMNP, 35)


# speedup vs baseline: 1.3217x; 1.3217x over previous
import functools

import jax
import jax.numpy as jnp
from jax import lax
from jax.experimental import pallas as pl
from jax.experimental.pallas import tpu as pltpu
from jax.experimental.pallas import tpu_sc as plsc

N = 100000
C = 32
SUMNP = 1600000
NCLUST = 4096
FULLSCALE = 14.0
SCALE = 50.0

NW = 32
CPW = NCLUST // NW
SPW = SUMNP // NW
CH = 128
W2 = 400
NST = SPW // W2
BIG = 3.0e38

_mesh = plsc.VectorSubcoreMesh(core_axis_name="c", subcore_axis_name="s")
_cparams = pltpu.CompilerParams(use_tc_tiling_on_sc=False,
                                needs_layout_passes=False)


def _wid():
    return lax.axis_index("s") * 2 + lax.axis_index("c")


def _stats_body(pidx_hbm, coords_hbm, off_hbm, stats_hbm,
                off_v, idx_v, rows_v, stats_v, sem):
    w = _wid()
    cbase = w * CPW
    pltpu.sync_copy(off_hbm.at[pl.ds(cbase, CPW + 16)], off_v)

    @pl.loop(0, CPW)
    def _cluster(c):
        vv = off_v[pl.ds(c, 16)]
        start = vv[0]
        end = vv[1]
        s_al = (start // 8) * 8
        nch = (end - s_al + CH - 1) // CH

        def chunk(k, carry):
            acc_s, acc_mn, acc_mx = carry
            pos = s_al + k * CH
            pltpu.sync_copy(pidx_hbm.at[pl.ds(pos, CH)], idx_v)
            pltpu.async_copy(coords_hbm.at[idx_v], rows_v, sem).wait()
            lo = jnp.maximum(start - pos, 0)
            hi = jnp.minimum(end - pos, CH)

            def point(j, pc):
                s2, mn2, mx2 = pc
                v = rows_v[j]
                return (s2 + v, jnp.minimum(mn2, v), jnp.maximum(mx2, v))

            return lax.fori_loop(lo, hi, point, (acc_s, acc_mn, acc_mx))

        init = (jnp.zeros((16,), jnp.float32),
                jnp.full((16,), BIG, jnp.float32),
                jnp.full((16,), -BIG, jnp.float32))
        acc_s, acc_mn, acc_mx = lax.fori_loop(0, nch, chunk, init)
        stats_v[c, pl.ds(0, 16)] = acc_s
        stats_v[c, pl.ds(16, 16)] = acc_mn
        stats_v[c, pl.ds(32, 16)] = acc_mx

    pltpu.sync_copy(stats_v, stats_hbm.at[pl.ds(cbase, CPW)])


@jax.jit
def _sc_stats(pidx_pad, coords_pad, off_pad):
    k = pl.kernel(
        _stats_body,
        out_type=jax.ShapeDtypeStruct((NCLUST, 48), jnp.float32),
        mesh=_mesh,
        scratch_types=[
            pltpu.VMEM((CPW + 16,), jnp.int32),
            pltpu.VMEM((CH,), jnp.int32),
            pltpu.VMEM((CH, 16), jnp.float32),
            pltpu.VMEM((CPW, 48), jnp.float32),
            pltpu.SemaphoreType.DMA,
        ],
        compiler_params=_cparams,
    )
    return k(pidx_pad, coords_pad, off_pad)


def _params_body(stats_ref, lo_ref, hi_ref, r_ref, out_ref):
    stats = stats_ref[...]
    cnt = jnp.maximum((hi_ref[...] - lo_ref[...]).astype(jnp.float32), 1.0)
    mean = stats[:, 0:3] / cnt
    cmin = stats[:, 16:19] - mean
    cmax = stats[:, 32:35] - mean
    ext = jnp.max(cmax - cmin, axis=1, keepdims=True)
    scale = jnp.minimum(1.0 / jnp.maximum(ext / FULLSCALE, 1e-6) - 0.01, SCALE)
    min_xyz = cmin * scale
    max_xyz = cmax * scale
    rng = max_xyz - min_xyz
    r0 = r_ref[0:1, :]
    r1 = r_ref[1:2, :]
    off = (-min_xyz
           + jnp.maximum(FULLSCALE - rng - 0.001, 0.0) * r0
           + jnp.minimum(FULLSCALE - rng + 0.001, 0.0) * r1)
    b = off - mean * scale
    out_ref[...] = jnp.concatenate([scale, b], axis=1)


@jax.jit
def _tc_params(stats, off_lo, off_hi, r):
    return pl.pallas_call(
        _params_body,
        out_shape=jax.ShapeDtypeStruct((NCLUST, 4), jnp.float32),
    )(stats, off_lo, off_hi, r)


_QCHUNKS = ((0, 128), (128, 128), (256, 128), (384, 16))


def _apply_body(pidx_hbm, cid_hbm, t_hbm, p_hbm, out_hbm,
                idx0, idx1, cid0, cid1, g0, g1, o0, o1, ptab,
                sg0, sg1, so0, so1):
    idx = (idx0, idx1)
    cidv = (cid0, cid1)
    g = (g0, g1)
    o = (o0, o1)
    sg = (sg0, sg1)
    so = (so0, so1)
    w = _wid()
    base0 = w * SPW
    pltpu.sync_copy(p_hbm, ptab)
    lane = lax.iota(jnp.int32, 16)
    sp0 = jnp.zeros((16,), jnp.int32)
    sp1 = jnp.full((16,), 1, jnp.int32)
    sp2 = jnp.full((16,), 2, jnp.int32)
    sp3 = jnp.full((16,), 3, jnp.int32)

    def fetch(s, b):
        base = base0 + s * W2
        pltpu.sync_copy(pidx_hbm.at[pl.ds(base, W2)], idx[b])
        pltpu.sync_copy(cid_hbm.at[pl.ds(base, W2)], cidv[b])
        for q0, qn in _QCHUNKS:
            pltpu.async_copy(t_hbm.at[idx[b].at[pl.ds(q0, qn)]],
                             g[b].at[pl.ds(q0, qn)], sg[b])

    def drain(b):
        for q0, qn in _QCHUNKS:
            pltpu.make_async_copy(t_hbm.at[idx[b].at[pl.ds(q0, qn)]],
                                  g[b].at[pl.ds(q0, qn)], sg[b]).wait()

    def compute(b):
        gv = g[b]
        cv = cidv[b]
        ov = o[b]

        @plsc.parallel_loop(0, W2 // 16, unroll=2)
        def _blk(q):
            j = q * 16
            rows35 = (j + lane) * 35
            cids = cv[pl.ds(j, 16)]
            s = plsc.load_gather(ptab, [cids, sp0])
            b0 = plsc.load_gather(ptab, [cids, sp1])
            b1 = plsc.load_gather(ptab, [cids, sp2])
            b2 = plsc.load_gather(ptab, [cids, sp3])
            rows = j + lane
            cx = plsc.load_gather(gv, [rows, sp0])
            cy = plsc.load_gather(gv, [rows, sp1])
            cz = plsc.load_gather(gv, [rows, sp2])
            plsc.store_scatter(ov, [rows35], cx * s + b0)
            plsc.store_scatter(ov, [rows35 + 1], cy * s + b1)
            plsc.store_scatter(ov, [rows35 + 2], cz * s + b2)

        @plsc.parallel_loop(0, W2, unroll=4)
        def _pt(j):
            f0 = gv[j, pl.ds(8, 16)]
            f1 = gv[j, pl.ds(24, 16)]
            plsc.store_scatter(ov, [j * 35 + 3 + lane], f0)
            plsc.store_scatter(ov, [j * 35 + 19 + lane], f1)

    def out_issue(s, b):
        base = base0 + s * W2
        return (pltpu.async_copy(o[b].at[pl.ds(0, W2 * 35)],
                                 out_hbm.at[pl.ds(base * 35, W2 * 35)],
                                 so[b]),)

    fetch(0, 0)
    fetch(1, 1)

    @pl.loop(0, NST // 2)
    def _pair(pr):
        s0 = 2 * pr
        drain(0)
        compute(0)
        cp0 = out_issue(s0, 0)
        fetch(s0 + 2, 0)
        drain(1)
        compute(1)
        cp1 = out_issue(s0 + 1, 1)
        fetch(s0 + 3, 1)
        for cp in cp0 + cp1:
            cp.wait()

    drain(0)
    compute(0)
    for cp in out_issue(NST - 1, 0):
        cp.wait()
    drain(1)


@jax.jit
def _sc_apply(pidx, cid, t, params):
    k = pl.kernel(
        _apply_body,
        out_type=jax.ShapeDtypeStruct((SUMNP * 35,), jnp.float32),
        mesh=_mesh,
        scratch_types=(
            [pltpu.VMEM((W2,), jnp.int32)] * 4
            + [pltpu.VMEM((W2, 48), jnp.float32),
               pltpu.VMEM((W2, 48), jnp.float32),
               pltpu.VMEM((W2 * 35 + 16,), jnp.float32),
               pltpu.VMEM((W2 * 35 + 16,), jnp.float32),
               pltpu.VMEM((NCLUST, 4), jnp.float32)]
            + [pltpu.SemaphoreType.DMA] * 4
        ),
        compiler_params=_cparams,
    )
    return k(pidx, cid, t, params)


def kernel(feats, coords, cluster_ids, point_idxs, clusters_offset):
    z13 = jnp.zeros((N, 13), jnp.float32)
    coords_pad = jnp.concatenate([coords, z13], axis=1)
    t = jnp.concatenate([coords, jnp.zeros((N, 5), jnp.float32),
                         feats, jnp.zeros((N, 8), jnp.float32)], axis=1)
    pidx = point_idxs.astype(jnp.int32)
    cid = cluster_ids.astype(jnp.int32)
    offs = clusters_offset.astype(jnp.int32)
    pidx_pad = jnp.concatenate([pidx, jnp.zeros((512,), jnp.int32)])
    cid_pad = jnp.concatenate([cid, jnp.zeros((512,), jnp.int32)])
    off_pad = jnp.concatenate(
        [offs, jnp.full((15,), SUMNP, jnp.int32)])

    stats = _sc_stats(pidx_pad, coords_pad, off_pad)
    r = jax.random.uniform(jax.random.key(42), (2, 3), dtype=jnp.float32)
    params = _tc_params(stats,
                        offs[:-1].reshape(NCLUST, 1),
                        offs[1:].reshape(NCLUST, 1), r)
    return _sc_apply(pidx_pad, cid_pad, t, params).reshape(SUMNP, 35)


# speedup vs baseline: 1.3320x; 1.0078x over previous
import functools

import jax
import jax.numpy as jnp
from jax import lax
from jax.experimental import pallas as pl
from jax.experimental.pallas import tpu as pltpu
from jax.experimental.pallas import tpu_sc as plsc

N = 100000
C = 32
SUMNP = 1600000
NCLUST = 4096
FULLSCALE = 14.0
SCALE = 50.0

NW = 32
CPW = NCLUST // NW
SPW = SUMNP // NW
CH = 128
W2 = 400
NST = SPW // W2
BIG = 3.0e38

_mesh = plsc.VectorSubcoreMesh(core_axis_name="c", subcore_axis_name="s")
_cparams = pltpu.CompilerParams(use_tc_tiling_on_sc=False,
                                needs_layout_passes=False)


def _wid():
    return lax.axis_index("s") * 2 + lax.axis_index("c")


def _stats_body(pidx_hbm, coords_hbm, off_hbm, stats_hbm,
                off_v, idx_v, rows_v, stats_v, sem):
    w = _wid()
    cbase = w * CPW
    pltpu.sync_copy(off_hbm.at[pl.ds(cbase, CPW + 16)], off_v)

    @pl.loop(0, CPW)
    def _cluster(c):
        vv = off_v[pl.ds(c, 16)]
        start = vv[0]
        end = vv[1]
        s_al = (start // 8) * 8
        nch = (end - s_al + CH - 1) // CH

        def chunk(k, carry):
            acc_s, acc_mn, acc_mx = carry
            pos = s_al + k * CH
            pltpu.sync_copy(pidx_hbm.at[pl.ds(pos, CH)], idx_v)
            pltpu.async_copy(coords_hbm.at[idx_v], rows_v, sem).wait()
            lo = jnp.maximum(start - pos, 0)
            hi = jnp.minimum(end - pos, CH)

            def point(j, pc):
                s2, mn2, mx2 = pc
                v = rows_v[j]
                return (s2 + v, jnp.minimum(mn2, v), jnp.maximum(mx2, v))

            return lax.fori_loop(lo, hi, point, (acc_s, acc_mn, acc_mx))

        init = (jnp.zeros((16,), jnp.float32),
                jnp.full((16,), BIG, jnp.float32),
                jnp.full((16,), -BIG, jnp.float32))
        acc_s, acc_mn, acc_mx = lax.fori_loop(0, nch, chunk, init)
        stats_v[c, pl.ds(0, 16)] = acc_s
        stats_v[c, pl.ds(16, 16)] = acc_mn
        stats_v[c, pl.ds(32, 16)] = acc_mx

    pltpu.sync_copy(stats_v, stats_hbm.at[pl.ds(cbase, CPW)])


@jax.jit
def _sc_stats(pidx_pad, coords_pad, off_pad):
    k = pl.kernel(
        _stats_body,
        out_type=jax.ShapeDtypeStruct((NCLUST, 48), jnp.float32),
        mesh=_mesh,
        scratch_types=[
            pltpu.VMEM((CPW + 16,), jnp.int32),
            pltpu.VMEM((CH,), jnp.int32),
            pltpu.VMEM((CH, 16), jnp.float32),
            pltpu.VMEM((CPW, 48), jnp.float32),
            pltpu.SemaphoreType.DMA,
        ],
        compiler_params=_cparams,
    )
    return k(pidx_pad, coords_pad, off_pad)


def _params_body(stats_ref, lo_ref, hi_ref, r_ref, out_ref):
    stats = stats_ref[...]
    cnt = jnp.maximum((hi_ref[...] - lo_ref[...]).astype(jnp.float32), 1.0)
    mean = stats[:, 0:3] / cnt
    cmin = stats[:, 16:19] - mean
    cmax = stats[:, 32:35] - mean
    ext = jnp.max(cmax - cmin, axis=1, keepdims=True)
    scale = jnp.minimum(1.0 / jnp.maximum(ext / FULLSCALE, 1e-6) - 0.01, SCALE)
    min_xyz = cmin * scale
    max_xyz = cmax * scale
    rng = max_xyz - min_xyz
    r0 = r_ref[0:1, :]
    r1 = r_ref[1:2, :]
    off = (-min_xyz
           + jnp.maximum(FULLSCALE - rng - 0.001, 0.0) * r0
           + jnp.minimum(FULLSCALE - rng + 0.001, 0.0) * r1)
    b = off - mean * scale
    out_ref[...] = jnp.concatenate([scale, b], axis=1)


@jax.jit
def _tc_params(stats, off_lo, off_hi, r):
    return pl.pallas_call(
        _params_body,
        out_shape=jax.ShapeDtypeStruct((NCLUST, 4), jnp.float32),
    )(stats, off_lo, off_hi, r)


_QCHUNKS = ((0, 128), (128, 128), (256, 128), (384, 16))


def _apply_body(pidx_hbm, cid_hbm, t_hbm, p_hbm, out_hbm,
                idx0, idx1, cid0, cid1, g0, g1, o0, o1, ptab,
                sg0, sg1, so0, so1):
    idx = (idx0, idx1)
    cidv = (cid0, cid1)
    g = (g0, g1)
    o = (o0, o1)
    sg = (sg0, sg1)
    so = (so0, so1)
    w = _wid()
    base0 = w * SPW
    pltpu.sync_copy(p_hbm, ptab)
    lane = lax.iota(jnp.int32, 16)
    sp0 = jnp.zeros((16,), jnp.int32)
    sp1 = jnp.full((16,), 1, jnp.int32)
    sp2 = jnp.full((16,), 2, jnp.int32)
    sp3 = jnp.full((16,), 3, jnp.int32)

    def fetch(s, b):
        base = base0 + s * W2
        pltpu.sync_copy(pidx_hbm.at[pl.ds(base, W2)], idx[b])
        pltpu.sync_copy(cid_hbm.at[pl.ds(base, W2)], cidv[b])
        for q0, qn in _QCHUNKS:
            pltpu.async_copy(t_hbm.at[idx[b].at[pl.ds(q0, qn)]],
                             g[b].at[pl.ds(q0, qn)], sg[b])

    def drain(b):
        for q0, qn in _QCHUNKS:
            pltpu.make_async_copy(t_hbm.at[idx[b].at[pl.ds(q0, qn)]],
                                  g[b].at[pl.ds(q0, qn)], sg[b]).wait()

    def compute(b):
        gv = g[b]
        cv = cidv[b]
        ov = o[b]

        @plsc.parallel_loop(0, W2 // 16, unroll=2)
        def _blk(q):
            j = q * 16
            rows35 = (j + lane) * 35
            cids = cv[pl.ds(j, 16)]
            s = plsc.load_gather(ptab, [cids, sp0])
            b0 = plsc.load_gather(ptab, [cids, sp1])
            b1 = plsc.load_gather(ptab, [cids, sp2])
            b2 = plsc.load_gather(ptab, [cids, sp3])
            rows = j + lane
            cx = plsc.load_gather(gv, [rows, sp0])
            cy = plsc.load_gather(gv, [rows, sp1])
            cz = plsc.load_gather(gv, [rows, sp2])
            plsc.store_scatter(ov, [rows35], cx * s + b0)
            plsc.store_scatter(ov, [rows35 + 1], cy * s + b1)
            plsc.store_scatter(ov, [rows35 + 2], cz * s + b2)

        @plsc.parallel_loop(0, W2, unroll=4)
        def _pt(j):
            f0 = gv[j, pl.ds(8, 16)]
            f1 = gv[j, pl.ds(24, 16)]
            plsc.store_scatter(ov, [j * 35 + 3 + lane], f0)
            plsc.store_scatter(ov, [j * 35 + 19 + lane], f1)

    def out_issue(s, b):
        base = base0 + s * W2
        return (pltpu.async_copy(o[b].at[pl.ds(0, W2 * 35)],
                                 out_hbm.at[pl.ds(base * 35, W2 * 35)],
                                 so[b]),)

    fetch(0, 0)
    fetch(1, 1)

    @pl.loop(0, NST // 2)
    def _pair(pr):
        s0 = 2 * pr
        drain(0)
        compute(0)
        cp0 = out_issue(s0, 0)
        fetch(s0 + 2, 0)
        drain(1)
        compute(1)
        cp1 = out_issue(s0 + 1, 1)
        fetch(s0 + 3, 1)
        for cp in cp0 + cp1:
            cp.wait()

    drain(0)
    compute(0)
    for cp in out_issue(NST - 1, 0):
        cp.wait()
    drain(1)


@jax.jit
def _sc_apply(pidx, cid, t, params):
    k = pl.kernel(
        _apply_body,
        out_type=jax.ShapeDtypeStruct((SUMNP * 35,), jnp.float32),
        mesh=_mesh,
        scratch_types=(
            [pltpu.VMEM((W2,), jnp.int32)] * 4
            + [pltpu.VMEM((W2, 40), jnp.float32),
               pltpu.VMEM((W2, 40), jnp.float32),
               pltpu.VMEM((W2 * 35 + 16,), jnp.float32),
               pltpu.VMEM((W2 * 35 + 16,), jnp.float32),
               pltpu.VMEM((NCLUST, 4), jnp.float32)]
            + [pltpu.SemaphoreType.DMA] * 4
        ),
        compiler_params=_cparams,
    )
    return k(pidx, cid, t, params)


def kernel(feats, coords, cluster_ids, point_idxs, clusters_offset):
    z13 = jnp.zeros((N, 13), jnp.float32)
    coords_pad = jnp.concatenate([coords, z13], axis=1)
    t = jnp.concatenate([coords, jnp.zeros((N, 5), jnp.float32),
                         feats], axis=1)
    pidx = point_idxs.astype(jnp.int32)
    cid = cluster_ids.astype(jnp.int32)
    offs = clusters_offset.astype(jnp.int32)
    pidx_pad = jnp.concatenate([pidx, jnp.zeros((512,), jnp.int32)])
    cid_pad = jnp.concatenate([cid, jnp.zeros((512,), jnp.int32)])
    off_pad = jnp.concatenate(
        [offs, jnp.full((15,), SUMNP, jnp.int32)])

    stats = _sc_stats(pidx_pad, coords_pad, off_pad)
    r = jax.random.uniform(jax.random.key(42), (2, 3), dtype=jnp.float32)
    params = _tc_params(stats,
                        offs[:-1].reshape(NCLUST, 1),
                        offs[1:].reshape(NCLUST, 1), r)
    return _sc_apply(pidx_pad, cid_pad, t, params).reshape(SUMNP, 35)


# speedup vs baseline: 1.3551x; 1.0173x over previous
import functools

import jax
import jax.numpy as jnp
from jax import lax
from jax.experimental import pallas as pl
from jax.experimental.pallas import tpu as pltpu
from jax.experimental.pallas import tpu_sc as plsc

N = 100000
C = 32
SUMNP = 1600000
NCLUST = 4096
FULLSCALE = 14.0
SCALE = 50.0

NW = 32
CPW = NCLUST // NW
SPW = SUMNP // NW
CH = 128
W2 = 400
NST = SPW // W2
BIG = 3.0e38

_mesh = plsc.VectorSubcoreMesh(core_axis_name="c", subcore_axis_name="s")
_cparams = pltpu.CompilerParams(use_tc_tiling_on_sc=False,
                                needs_layout_passes=False)


def _wid():
    return lax.axis_index("s") * 2 + lax.axis_index("c")


def _stats_body(pidx_hbm, coords_hbm, off_hbm, stats_hbm,
                off_v, idx_v, rows_v, stats_v, sem):
    w = _wid()
    cbase = w * CPW
    pltpu.sync_copy(off_hbm.at[pl.ds(cbase, CPW + 16)], off_v)

    @pl.loop(0, CPW)
    def _cluster(c):
        vv = off_v[pl.ds(c, 16)]
        start = vv[0]
        end = vv[1]
        s_al = (start // 8) * 8
        nch = (end - s_al + CH - 1) // CH

        def chunk(k, carry):
            acc_s, acc_mn, acc_mx = carry
            pos = s_al + k * CH
            pltpu.sync_copy(pidx_hbm.at[pl.ds(pos, CH)], idx_v)
            pltpu.async_copy(coords_hbm.at[idx_v], rows_v, sem).wait()
            lo = jnp.maximum(start - pos, 0)
            hi = jnp.minimum(end - pos, CH)

            def point(j, pc):
                s2, mn2, mx2 = pc
                v = rows_v[j]
                return (s2 + v, jnp.minimum(mn2, v), jnp.maximum(mx2, v))

            return lax.fori_loop(lo, hi, point, (acc_s, acc_mn, acc_mx))

        init = (jnp.zeros((16,), jnp.float32),
                jnp.full((16,), BIG, jnp.float32),
                jnp.full((16,), -BIG, jnp.float32))
        acc_s, acc_mn, acc_mx = lax.fori_loop(0, nch, chunk, init)
        stats_v[c, pl.ds(0, 16)] = acc_s
        stats_v[c, pl.ds(16, 16)] = acc_mn
        stats_v[c, pl.ds(32, 16)] = acc_mx

    pltpu.sync_copy(stats_v, stats_hbm.at[pl.ds(cbase, CPW)])


@jax.jit
def _sc_stats(pidx_pad, coords_pad, off_pad):
    k = pl.kernel(
        _stats_body,
        out_type=jax.ShapeDtypeStruct((NCLUST, 48), jnp.float32),
        mesh=_mesh,
        scratch_types=[
            pltpu.VMEM((CPW + 16,), jnp.int32),
            pltpu.VMEM((CH,), jnp.int32),
            pltpu.VMEM((CH, 16), jnp.float32),
            pltpu.VMEM((CPW, 48), jnp.float32),
            pltpu.SemaphoreType.DMA,
        ],
        compiler_params=_cparams,
    )
    return k(pidx_pad, coords_pad, off_pad)


def _params_body(stats_ref, lo_ref, hi_ref, r_ref, out_ref):
    stats = stats_ref[...]
    cnt = jnp.maximum((hi_ref[...] - lo_ref[...]).astype(jnp.float32), 1.0)
    mean = stats[:, 0:3] / cnt
    cmin = stats[:, 16:19] - mean
    cmax = stats[:, 32:35] - mean
    ext = jnp.max(cmax - cmin, axis=1, keepdims=True)
    scale = jnp.minimum(1.0 / jnp.maximum(ext / FULLSCALE, 1e-6) - 0.01, SCALE)
    min_xyz = cmin * scale
    max_xyz = cmax * scale
    rng = max_xyz - min_xyz
    r0 = r_ref[0:1, :]
    r1 = r_ref[1:2, :]
    off = (-min_xyz
           + jnp.maximum(FULLSCALE - rng - 0.001, 0.0) * r0
           + jnp.minimum(FULLSCALE - rng + 0.001, 0.0) * r1)
    b = off - mean * scale
    out_ref[...] = jnp.concatenate([scale, b], axis=1)


@jax.jit
def _tc_params(stats, off_lo, off_hi, r):
    return pl.pallas_call(
        _params_body,
        out_shape=jax.ShapeDtypeStruct((NCLUST, 4), jnp.float32),
    )(stats, off_lo, off_hi, r)


_QCHUNKS = ((0, 128), (128, 128), (256, 128), (384, 16))


def _apply_body(ic_hbm, t_hbm, p_hbm, out_hbm,
                ic0, ic1, g0, g1, o0, o1, ptab,
                sg0, sg1, so0, so1):
    ic = (ic0, ic1)
    g = (g0, g1)
    o = (o0, o1)
    sg = (sg0, sg1)
    so = (so0, so1)
    w = _wid()
    base0 = w * SPW
    pltpu.sync_copy(p_hbm, ptab)
    lane = lax.iota(jnp.int32, 16)
    sp0 = jnp.zeros((16,), jnp.int32)
    sp1 = jnp.full((16,), 1, jnp.int32)
    sp2 = jnp.full((16,), 2, jnp.int32)
    sp3 = jnp.full((16,), 3, jnp.int32)

    def fetch(s, b):
        base = base0 + s * W2
        pltpu.sync_copy(ic_hbm.at[:, pl.ds(base, W2)], ic[b])
        for q0, qn in _QCHUNKS:
            pltpu.async_copy(t_hbm.at[ic[b].at[0, pl.ds(q0, qn)]],
                             g[b].at[pl.ds(q0, qn)], sg[b])

    def drain(b):
        for q0, qn in _QCHUNKS:
            pltpu.make_async_copy(t_hbm.at[ic[b].at[0, pl.ds(q0, qn)]],
                                  g[b].at[pl.ds(q0, qn)], sg[b]).wait()

    def compute(b):
        gv = g[b]
        cv = ic[b]
        ov = o[b]

        @plsc.parallel_loop(0, W2 // 16, unroll=2)
        def _blk(q):
            j = q * 16
            rows35 = (j + lane) * 35
            cids = cv[1, pl.ds(j, 16)]
            s = plsc.load_gather(ptab, [cids, sp0])
            b0 = plsc.load_gather(ptab, [cids, sp1])
            b1 = plsc.load_gather(ptab, [cids, sp2])
            b2 = plsc.load_gather(ptab, [cids, sp3])
            rows = j + lane
            cx = plsc.load_gather(gv, [rows, sp0])
            cy = plsc.load_gather(gv, [rows, sp1])
            cz = plsc.load_gather(gv, [rows, sp2])
            plsc.store_scatter(ov, [rows35], cx * s + b0)
            plsc.store_scatter(ov, [rows35 + 1], cy * s + b1)
            plsc.store_scatter(ov, [rows35 + 2], cz * s + b2)

        @plsc.parallel_loop(0, W2, unroll=4)
        def _pt(j):
            f0 = gv[j, pl.ds(8, 16)]
            f1 = gv[j, pl.ds(24, 16)]
            plsc.store_scatter(ov, [j * 35 + 3 + lane], f0)
            plsc.store_scatter(ov, [j * 35 + 19 + lane], f1)

    def out_issue(s, b):
        base = base0 + s * W2
        return (pltpu.async_copy(o[b].at[pl.ds(0, W2 * 35)],
                                 out_hbm.at[pl.ds(base * 35, W2 * 35)],
                                 so[b]),)

    fetch(0, 0)
    fetch(1, 1)

    @pl.loop(0, NST // 2)
    def _pair(pr):
        s0 = 2 * pr
        drain(0)
        compute(0)
        cp0 = out_issue(s0, 0)
        fetch(s0 + 2, 0)
        drain(1)
        compute(1)
        cp1 = out_issue(s0 + 1, 1)
        fetch(s0 + 3, 1)
        for cp in cp0 + cp1:
            cp.wait()

    drain(0)
    compute(0)
    for cp in out_issue(NST - 1, 0):
        cp.wait()
    drain(1)


@jax.jit
def _sc_apply(ic, t, params):
    k = pl.kernel(
        _apply_body,
        out_type=jax.ShapeDtypeStruct((SUMNP * 35,), jnp.float32),
        mesh=_mesh,
        scratch_types=(
            [pltpu.VMEM((2, W2), jnp.int32)] * 2
            + [pltpu.VMEM((W2, 40), jnp.float32),
               pltpu.VMEM((W2, 40), jnp.float32),
               pltpu.VMEM((W2 * 35 + 16,), jnp.float32),
               pltpu.VMEM((W2 * 35 + 16,), jnp.float32),
               pltpu.VMEM((NCLUST, 4), jnp.float32)]
            + [pltpu.SemaphoreType.DMA] * 4
        ),
        compiler_params=_cparams,
    )
    return k(ic, t, params)


def kernel(feats, coords, cluster_ids, point_idxs, clusters_offset):
    z13 = jnp.zeros((N, 13), jnp.float32)
    coords_pad = jnp.concatenate([coords, z13], axis=1)
    t = jnp.concatenate([coords, jnp.zeros((N, 5), jnp.float32),
                         feats], axis=1)
    pidx = point_idxs.astype(jnp.int32)
    cid = cluster_ids.astype(jnp.int32)
    offs = clusters_offset.astype(jnp.int32)
    pidx_pad = jnp.concatenate([pidx, jnp.zeros((512,), jnp.int32)])
    cid_pad = jnp.concatenate([cid, jnp.zeros((512,), jnp.int32)])
    off_pad = jnp.concatenate(
        [offs, jnp.full((15,), SUMNP, jnp.int32)])

    stats = _sc_stats(pidx_pad, coords_pad, off_pad)
    r = jax.random.uniform(jax.random.key(42), (2, 3), dtype=jnp.float32)
    params = _tc_params(stats,
                        offs[:-1].reshape(NCLUST, 1),
                        offs[1:].reshape(NCLUST, 1), r)
    ic = jnp.stack([pidx_pad, cid_pad])
    return _sc_apply(ic, t, params).reshape(SUMNP, 35)


# speedup vs baseline: 1.3551x; 1.0000x over previous
import functools

import jax
import jax.numpy as jnp
from jax import lax
from jax.experimental import pallas as pl
from jax.experimental.pallas import tpu as pltpu
from jax.experimental.pallas import tpu_sc as plsc

N = 100000
C = 32
SUMNP = 1600000
NCLUST = 4096
FULLSCALE = 14.0
SCALE = 50.0

NW = 32
CPW = NCLUST // NW
SPW = SUMNP // NW
CH = 128
W2 = 400
NST = SPW // W2
BIG = 3.0e38

_mesh = plsc.VectorSubcoreMesh(core_axis_name="c", subcore_axis_name="s")
_cparams = pltpu.CompilerParams(use_tc_tiling_on_sc=False,
                                needs_layout_passes=False)


def _wid():
    return lax.axis_index("s") * 2 + lax.axis_index("c")


def _stats_body(pidx_hbm, coords_hbm, off_hbm, stats_hbm,
                off_v, idx_v, rows_v, stats_v, sem):
    w = _wid()
    cbase = w * CPW
    pltpu.sync_copy(off_hbm.at[pl.ds(cbase, CPW + 16)], off_v)

    @pl.loop(0, CPW)
    def _cluster(c):
        vv = off_v[pl.ds(c, 16)]
        start = vv[0]
        end = vv[1]
        s_al = (start // 8) * 8
        nch = (end - s_al + CH - 1) // CH

        def chunk(k, carry):
            acc_s, acc_mn, acc_mx = carry
            pos = s_al + k * CH
            pltpu.sync_copy(pidx_hbm.at[pl.ds(pos, CH)], idx_v)
            pltpu.async_copy(coords_hbm.at[idx_v], rows_v, sem).wait()
            lo = jnp.maximum(start - pos, 0)
            hi = jnp.minimum(end - pos, CH)

            def point(j, pc):
                s2, mn2, mx2 = pc
                v = rows_v[j]
                return (s2 + v, jnp.minimum(mn2, v), jnp.maximum(mx2, v))

            return lax.fori_loop(lo, hi, point, (acc_s, acc_mn, acc_mx))

        init = (jnp.zeros((16,), jnp.float32),
                jnp.full((16,), BIG, jnp.float32),
                jnp.full((16,), -BIG, jnp.float32))
        acc_s, acc_mn, acc_mx = lax.fori_loop(0, nch, chunk, init)
        stats_v[c, pl.ds(0, 16)] = acc_s
        stats_v[c, pl.ds(16, 16)] = acc_mn
        stats_v[c, pl.ds(32, 16)] = acc_mx

    pltpu.sync_copy(stats_v, stats_hbm.at[pl.ds(cbase, CPW)])


@jax.jit
def _sc_stats(pidx_pad, coords_pad, off_pad):
    k = pl.kernel(
        _stats_body,
        out_type=jax.ShapeDtypeStruct((NCLUST, 48), jnp.float32),
        mesh=_mesh,
        scratch_types=[
            pltpu.VMEM((CPW + 16,), jnp.int32),
            pltpu.VMEM((CH,), jnp.int32),
            pltpu.VMEM((CH, 16), jnp.float32),
            pltpu.VMEM((CPW, 48), jnp.float32),
            pltpu.SemaphoreType.DMA,
        ],
        compiler_params=_cparams,
    )
    return k(pidx_pad, coords_pad, off_pad)


def _params_body(stats_ref, lo_ref, hi_ref, r_ref, out_ref):
    stats = stats_ref[...]
    cnt = jnp.maximum((hi_ref[...] - lo_ref[...]).astype(jnp.float32), 1.0)
    mean = stats[:, 0:3] / cnt
    cmin = stats[:, 16:19] - mean
    cmax = stats[:, 32:35] - mean
    ext = jnp.max(cmax - cmin, axis=1, keepdims=True)
    scale = jnp.minimum(1.0 / jnp.maximum(ext / FULLSCALE, 1e-6) - 0.01, SCALE)
    min_xyz = cmin * scale
    max_xyz = cmax * scale
    rng = max_xyz - min_xyz
    r0 = r_ref[0:1, :]
    r1 = r_ref[1:2, :]
    off = (-min_xyz
           + jnp.maximum(FULLSCALE - rng - 0.001, 0.0) * r0
           + jnp.minimum(FULLSCALE - rng + 0.001, 0.0) * r1)
    b = off - mean * scale
    out_ref[...] = jnp.concatenate([scale, b], axis=1)


@jax.jit
def _tc_params(stats, off_lo, off_hi, r):
    return pl.pallas_call(
        _params_body,
        out_shape=jax.ShapeDtypeStruct((NCLUST, 4), jnp.float32),
    )(stats, off_lo, off_hi, r)


_QCHUNKS = ((0, 80), (80, 80), (160, 80), (240, 80), (320, 80))


def _apply_body(ic_hbm, t_hbm, p_hbm, out_hbm,
                ic0, ic1, g0, g1, o0, o1, ptab,
                sg0, sg1, so0, so1):
    ic = (ic0, ic1)
    g = (g0, g1)
    o = (o0, o1)
    sg = (sg0, sg1)
    so = (so0, so1)
    w = _wid()
    base0 = w * SPW
    pltpu.sync_copy(p_hbm, ptab)
    lane = lax.iota(jnp.int32, 16)
    sp0 = jnp.zeros((16,), jnp.int32)
    sp1 = jnp.full((16,), 1, jnp.int32)
    sp2 = jnp.full((16,), 2, jnp.int32)
    sp3 = jnp.full((16,), 3, jnp.int32)

    def fetch(s, b):
        base = base0 + s * W2
        pltpu.sync_copy(ic_hbm.at[:, pl.ds(base, W2)], ic[b])
        for q0, qn in _QCHUNKS:
            pltpu.async_copy(t_hbm.at[ic[b].at[0, pl.ds(q0, qn)]],
                             g[b].at[pl.ds(q0, qn)], sg[b])

    def drain(b):
        for q0, qn in _QCHUNKS:
            pltpu.make_async_copy(t_hbm.at[ic[b].at[0, pl.ds(q0, qn)]],
                                  g[b].at[pl.ds(q0, qn)], sg[b]).wait()

    def compute(b):
        gv = g[b]
        cv = ic[b]
        ov = o[b]

        @plsc.parallel_loop(0, W2 // 16, unroll=2)
        def _blk(q):
            j = q * 16
            rows35 = (j + lane) * 35
            cids = cv[1, pl.ds(j, 16)]
            s = plsc.load_gather(ptab, [cids, sp0])
            b0 = plsc.load_gather(ptab, [cids, sp1])
            b1 = plsc.load_gather(ptab, [cids, sp2])
            b2 = plsc.load_gather(ptab, [cids, sp3])
            rows = j + lane
            cx = plsc.load_gather(gv, [rows, sp0])
            cy = plsc.load_gather(gv, [rows, sp1])
            cz = plsc.load_gather(gv, [rows, sp2])
            plsc.store_scatter(ov, [rows35], cx * s + b0)
            plsc.store_scatter(ov, [rows35 + 1], cy * s + b1)
            plsc.store_scatter(ov, [rows35 + 2], cz * s + b2)

        @plsc.parallel_loop(0, W2, unroll=4)
        def _pt(j):
            f0 = gv[j, pl.ds(8, 16)]
            f1 = gv[j, pl.ds(24, 16)]
            plsc.store_scatter(ov, [j * 35 + 3 + lane], f0)
            plsc.store_scatter(ov, [j * 35 + 19 + lane], f1)

    def out_issue(s, b):
        base = base0 + s * W2
        return (pltpu.async_copy(o[b].at[pl.ds(0, W2 * 35)],
                                 out_hbm.at[pl.ds(base * 35, W2 * 35)],
                                 so[b]),)

    fetch(0, 0)
    fetch(1, 1)

    @pl.loop(0, NST // 2)
    def _pair(pr):
        s0 = 2 * pr
        drain(0)
        compute(0)
        cp0 = out_issue(s0, 0)
        fetch(s0 + 2, 0)
        drain(1)
        compute(1)
        cp1 = out_issue(s0 + 1, 1)
        fetch(s0 + 3, 1)
        for cp in cp0 + cp1:
            cp.wait()

    drain(0)
    compute(0)
    for cp in out_issue(NST - 1, 0):
        cp.wait()
    drain(1)


@jax.jit
def _sc_apply(ic, t, params):
    k = pl.kernel(
        _apply_body,
        out_type=jax.ShapeDtypeStruct((SUMNP * 35,), jnp.float32),
        mesh=_mesh,
        scratch_types=(
            [pltpu.VMEM((2, W2), jnp.int32)] * 2
            + [pltpu.VMEM((W2, 40), jnp.float32),
               pltpu.VMEM((W2, 40), jnp.float32),
               pltpu.VMEM((W2 * 35 + 16,), jnp.float32),
               pltpu.VMEM((W2 * 35 + 16,), jnp.float32),
               pltpu.VMEM((NCLUST, 4), jnp.float32)]
            + [pltpu.SemaphoreType.DMA] * 4
        ),
        compiler_params=_cparams,
    )
    return k(ic, t, params)


def kernel(feats, coords, cluster_ids, point_idxs, clusters_offset):
    z13 = jnp.zeros((N, 13), jnp.float32)
    coords_pad = jnp.concatenate([coords, z13], axis=1)
    t = jnp.concatenate([coords, jnp.zeros((N, 5), jnp.float32),
                         feats], axis=1)
    pidx = point_idxs.astype(jnp.int32)
    cid = cluster_ids.astype(jnp.int32)
    offs = clusters_offset.astype(jnp.int32)
    pidx_pad = jnp.concatenate([pidx, jnp.zeros((512,), jnp.int32)])
    cid_pad = jnp.concatenate([cid, jnp.zeros((512,), jnp.int32)])
    off_pad = jnp.concatenate(
        [offs, jnp.full((15,), SUMNP, jnp.int32)])

    stats = _sc_stats(pidx_pad, coords_pad, off_pad)
    r = jax.random.uniform(jax.random.key(42), (2, 3), dtype=jnp.float32)
    params = _tc_params(stats,
                        offs[:-1].reshape(NCLUST, 1),
                        offs[1:].reshape(NCLUST, 1), r)
    ic = jnp.stack([pidx_pad, cid_pad])
    return _sc_apply(ic, t, params).reshape(SUMNP, 35)


# speedup vs baseline: 1.3552x; 1.0001x over previous
import jax
import jax.numpy as jnp
from jax import lax
from jax.experimental import pallas as pl
from jax.experimental.pallas import tpu as pltpu
from jax.experimental.pallas import tpu_sc as plsc

N = 100000
C = 32
SUMNP = 1600000
NCLUST = 4096
FULLSCALE = 14.0
SCALE = 50.0

NW = 32
CPW = NCLUST // NW
SPW = SUMNP // NW
CH = 128
W2 = 400
NST = SPW // W2
BIG = 3.0e38

_mesh = plsc.VectorSubcoreMesh(core_axis_name="c", subcore_axis_name="s")
_cparams = pltpu.CompilerParams(use_tc_tiling_on_sc=False,
                                needs_layout_passes=False)


def _wid():
    return lax.axis_index("s") * 2 + lax.axis_index("c")


def _stats_body(pidx_hbm, coords_hbm, off_hbm, stats_hbm,
                off_v, idx_v, rows_v, stats_v, sem):
    w = _wid()
    cbase = w * CPW
    pltpu.sync_copy(off_hbm.at[pl.ds(cbase, CPW + 16)], off_v)

    @pl.loop(0, CPW)
    def _cluster(c):
        vv = off_v[pl.ds(c, 16)]
        start = vv[0]
        end = vv[1]
        s_al = (start // 8) * 8
        nch = (end - s_al + CH - 1) // CH

        def chunk(k, carry):
            acc_s, acc_mn, acc_mx = carry
            pos = s_al + k * CH
            pltpu.sync_copy(pidx_hbm.at[pl.ds(pos, CH)], idx_v)
            pltpu.async_copy(coords_hbm.at[idx_v], rows_v, sem).wait()
            lo = jnp.maximum(start - pos, 0)
            hi = jnp.minimum(end - pos, CH)

            def point(j, pc):
                s2, mn2, mx2 = pc
                v = rows_v[j]
                return (s2 + v, jnp.minimum(mn2, v), jnp.maximum(mx2, v))

            return lax.fori_loop(lo, hi, point, (acc_s, acc_mn, acc_mx))

        init = (jnp.zeros((16,), jnp.float32),
                jnp.full((16,), BIG, jnp.float32),
                jnp.full((16,), -BIG, jnp.float32))
        acc_s, acc_mn, acc_mx = lax.fori_loop(0, nch, chunk, init)
        stats_v[c, pl.ds(0, 16)] = acc_s
        stats_v[c, pl.ds(16, 16)] = acc_mn
        stats_v[c, pl.ds(32, 16)] = acc_mx

    pltpu.sync_copy(stats_v, stats_hbm.at[pl.ds(cbase, CPW)])


@jax.jit
def _sc_stats(pidx_pad, coords_pad, off_pad):
    k = pl.kernel(
        _stats_body,
        out_type=jax.ShapeDtypeStruct((NCLUST, 48), jnp.float32),
        mesh=_mesh,
        scratch_types=[
            pltpu.VMEM((CPW + 16,), jnp.int32),
            pltpu.VMEM((CH,), jnp.int32),
            pltpu.VMEM((CH, 16), jnp.float32),
            pltpu.VMEM((CPW, 48), jnp.float32),
            pltpu.SemaphoreType.DMA,
        ],
        compiler_params=_cparams,
    )
    return k(pidx_pad, coords_pad, off_pad)


def _params_body(stats_ref, lo_ref, hi_ref, r_ref, out_ref):
    stats = stats_ref[...]
    cnt = jnp.maximum((hi_ref[...] - lo_ref[...]).astype(jnp.float32), 1.0)
    mean = stats[:, 0:3] / cnt
    cmin = stats[:, 16:19] - mean
    cmax = stats[:, 32:35] - mean
    ext = jnp.max(cmax - cmin, axis=1, keepdims=True)
    scale = jnp.minimum(1.0 / jnp.maximum(ext / FULLSCALE, 1e-6) - 0.01, SCALE)
    min_xyz = cmin * scale
    max_xyz = cmax * scale
    rng = max_xyz - min_xyz
    r0 = r_ref[0:1, :]
    r1 = r_ref[1:2, :]
    off = (-min_xyz
           + jnp.maximum(FULLSCALE - rng - 0.001, 0.0) * r0
           + jnp.minimum(FULLSCALE - rng + 0.001, 0.0) * r1)
    b = off - mean * scale
    out_ref[...] = jnp.concatenate([scale, b], axis=1)


@jax.jit
def _tc_params(stats, off_lo, off_hi, r):
    return pl.pallas_call(
        _params_body,
        out_shape=jax.ShapeDtypeStruct((NCLUST, 4), jnp.float32),
    )(stats, off_lo, off_hi, r)


_QCHUNKS = ((0, 80), (80, 80), (160, 80), (240, 80), (320, 80))


def _apply_body(ic_hbm, t_hbm, p_hbm, out_hbm,
                ic0, ic1, g0, g1, o0, o1, ptab,
                sg0, sg1, so0, so1):
    ic = (ic0, ic1)
    g = (g0, g1)
    o = (o0, o1)
    sg = (sg0, sg1)
    so = (so0, so1)
    w = _wid()
    base0 = w * SPW
    pltpu.sync_copy(p_hbm, ptab)
    lane = lax.iota(jnp.int32, 16)
    sp0 = jnp.zeros((16,), jnp.int32)
    sp1 = jnp.full((16,), 1, jnp.int32)
    sp2 = jnp.full((16,), 2, jnp.int32)
    sp3 = jnp.full((16,), 3, jnp.int32)

    def fetch(s, b):
        base = base0 + s * W2
        pltpu.sync_copy(ic_hbm.at[:, pl.ds(base, W2)], ic[b])
        for q0, qn in _QCHUNKS:
            pltpu.async_copy(t_hbm.at[ic[b].at[0, pl.ds(q0, qn)]],
                             g[b].at[pl.ds(q0, qn)], sg[b])

    def drain(b):
        for q0, qn in _QCHUNKS:
            pltpu.make_async_copy(t_hbm.at[ic[b].at[0, pl.ds(q0, qn)]],
                                  g[b].at[pl.ds(q0, qn)], sg[b]).wait()

    def compute(b):
        gv = g[b]
        cv = ic[b]
        ov = o[b]

        @plsc.parallel_loop(0, W2 // 16, unroll=2)
        def _blk(q):
            j = q * 16
            rows35 = (j + lane) * 35
            cids = cv[1, pl.ds(j, 16)]
            s = plsc.load_gather(ptab, [cids, sp0])
            b0 = plsc.load_gather(ptab, [cids, sp1])
            b1 = plsc.load_gather(ptab, [cids, sp2])
            b2 = plsc.load_gather(ptab, [cids, sp3])
            rows = j + lane
            cx = plsc.load_gather(gv, [rows, sp0])
            cy = plsc.load_gather(gv, [rows, sp1])
            cz = plsc.load_gather(gv, [rows, sp2])
            plsc.store_scatter(ov, [rows35], cx * s + b0)
            plsc.store_scatter(ov, [rows35 + 1], cy * s + b1)
            plsc.store_scatter(ov, [rows35 + 2], cz * s + b2)

        @plsc.parallel_loop(0, W2, unroll=4)
        def _pt(j):
            f0 = gv[j, pl.ds(8, 16)]
            f1 = gv[j, pl.ds(24, 16)]
            plsc.store_scatter(ov, [j * 35 + 3 + lane], f0)
            plsc.store_scatter(ov, [j * 35 + 19 + lane], f1)

    def out_issue(s, b):
        base = base0 + s * W2
        return (pltpu.async_copy(o[b].at[pl.ds(0, W2 * 35)],
                                 out_hbm.at[pl.ds(base * 35, W2 * 35)],
                                 so[b]),)

    fetch(0, 0)
    fetch(1, 1)

    @pl.loop(0, NST // 2)
    def _pair(pr):
        s0 = 2 * pr
        drain(0)
        compute(0)
        cp0 = out_issue(s0, 0)
        fetch(s0 + 2, 0)
        drain(1)
        compute(1)
        cp1 = out_issue(s0 + 1, 1)
        fetch(s0 + 3, 1)
        for cp in cp0 + cp1:
            cp.wait()

    drain(0)
    compute(0)
    for cp in out_issue(NST - 1, 0):
        cp.wait()
    drain(1)


@jax.jit
def _sc_apply(ic, t, params):
    k = pl.kernel(
        _apply_body,
        out_type=jax.ShapeDtypeStruct((SUMNP * 35,), jnp.float32),
        mesh=_mesh,
        scratch_types=(
            [pltpu.VMEM((2, W2), jnp.int32)] * 2
            + [pltpu.VMEM((W2, 40), jnp.float32),
               pltpu.VMEM((W2, 40), jnp.float32),
               pltpu.VMEM((W2 * 35 + 16,), jnp.float32),
               pltpu.VMEM((W2 * 35 + 16,), jnp.float32),
               pltpu.VMEM((NCLUST, 4), jnp.float32)]
            + [pltpu.SemaphoreType.DMA] * 4
        ),
        compiler_params=_cparams,
    )
    return k(ic, t, params)


def kernel(feats, coords, cluster_ids, point_idxs, clusters_offset):
    z13 = jnp.zeros((N, 13), jnp.float32)
    coords_pad = jnp.concatenate([coords, z13], axis=1)
    t = jnp.concatenate([coords, jnp.zeros((N, 5), jnp.float32),
                         feats], axis=1)
    pidx = point_idxs.astype(jnp.int32)
    cid = cluster_ids.astype(jnp.int32)
    offs = clusters_offset.astype(jnp.int32)
    pidx_pad = jnp.concatenate([pidx, jnp.zeros((512,), jnp.int32)])
    cid_pad = jnp.concatenate([cid, jnp.zeros((512,), jnp.int32)])
    off_pad = jnp.concatenate(
        [offs, jnp.full((15,), SUMNP, jnp.int32)])

    stats = _sc_stats(pidx_pad, coords_pad, off_pad)
    r = jax.random.uniform(jax.random.key(42), (2, 3), dtype=jnp.float32)
    params = _tc_params(stats,
                        offs[:-1].reshape(NCLUST, 1),
                        offs[1:].reshape(NCLUST, 1), r)
    ic = jnp.stack([pidx_pad, cid_pad])
    return _sc_apply(ic, t, params).reshape(SUMNP, 35)
